# speedup vs baseline: 5.5276x; 5.5276x over previous
import functools
import jax
import jax.numpy as jnp
from jax.experimental import pallas as pl
from jax.experimental.pallas import tpu as pltpu

_BATCH = 256


def _tc_body(nodes_ref, idx_ref, w_ref, b_ref, out_ref, acc_ref):
    g = pl.program_id(0)
    ng = pl.num_programs(0)

    @pl.when(g == 0)
    def _init():
        acc_ref[...] = jnp.zeros_like(acc_ref)

    blk = nodes_ref[...]
    y = jnp.dot(blk, w_ref[...], preferred_element_type=jnp.float32)
    idx = idx_ref[0, 0, :]
    onehot = (idx[:, None] == jax.lax.broadcasted_iota(
        jnp.int32, (1, _BATCH), 1)).astype(jnp.float32)
    y1 = jnp.concatenate([y, jnp.ones_like(y)], axis=1)
    acc_ref[...] += jax.lax.dot_general(
        y1, onehot, dimension_numbers=(((0,), (0,)), ((), ())),
        preferred_element_type=jnp.float32)

    @pl.when(g == ng - 1)
    def _final():
        ys = acc_ref[0, :]
        cnt = acc_ref[1, :]
        logits = ys / jnp.maximum(cnt, 1.0) + b_ref[0, 0]
        out_ref[...] = jax.nn.sigmoid(logits)[:, None]


def kernel(nodes, edges, receivers, senders, global_latent, node_graph_idx,
           edge_graph_idx, W, b):
    n, d = nodes.shape
    br = 1000
    grid = n // br
    idx3 = node_graph_idx.reshape(grid, 1, br)
    b2 = b.reshape(1, 1)

    out = pl.pallas_call(
        _tc_body,
        grid=(grid,),
        in_specs=[
            pl.BlockSpec((br, d), lambda g: (g, 0)),
            pl.BlockSpec((1, 1, br), lambda g: (g, 0, 0)),
            pl.BlockSpec((d, 1), lambda g: (0, 0)),
            pl.BlockSpec((1, 1), lambda g: (0, 0)),
        ],
        out_specs=pl.BlockSpec((_BATCH, 1), lambda g: (0, 0)),
        out_shape=jax.ShapeDtypeStruct((_BATCH, 1), jnp.float32),
        scratch_shapes=[pltpu.VMEM((2, _BATCH), jnp.float32)],
    )(nodes, idx3, W, b2)
    return out


# speedup vs baseline: 11.9538x; 2.1626x over previous
import functools
import jax
import jax.numpy as jnp
from jax.experimental import pallas as pl
from jax.experimental.pallas import tpu as pltpu

_BATCH = 256


def _tc_body(nodes_ref, idx_ref, w_ref, b_ref, out_ref, acc_ref):
    g = pl.program_id(0)
    ng = pl.num_programs(0)

    @pl.when(g == 0)
    def _init():
        acc_ref[...] = jnp.zeros_like(acc_ref)

    blk = nodes_ref[...]
    y = jnp.dot(blk, w_ref[...], preferred_element_type=jnp.float32)
    idx = idx_ref[0, 0, :]
    onehot = (idx[:, None] == jax.lax.broadcasted_iota(
        jnp.int32, (1, _BATCH), 1)).astype(jnp.float32)
    y1 = jnp.concatenate([y, jnp.ones_like(y)], axis=1)
    acc_ref[...] += jax.lax.dot_general(
        y1, onehot, dimension_numbers=(((0,), (0,)), ((), ())),
        preferred_element_type=jnp.float32)

    @pl.when(g == ng - 1)
    def _final():
        ys = acc_ref[0, :]
        cnt = acc_ref[1, :]
        logits = ys / jnp.maximum(cnt, 1.0) + b_ref[0, 0]
        out_ref[...] = jax.nn.sigmoid(logits)[:, None]


def kernel(nodes, edges, receivers, senders, global_latent, node_graph_idx,
           edge_graph_idx, W, b):
    n, d = nodes.shape
    br = 4000
    grid = n // br
    idx3 = node_graph_idx.reshape(grid, 1, br)
    b2 = b.reshape(1, 1)

    out = pl.pallas_call(
        _tc_body,
        grid=(grid,),
        in_specs=[
            pl.BlockSpec((br, d), lambda g: (g, 0)),
            pl.BlockSpec((1, 1, br), lambda g: (g, 0, 0)),
            pl.BlockSpec((d, 1), lambda g: (0, 0)),
            pl.BlockSpec((1, 1), lambda g: (0, 0)),
        ],
        out_specs=pl.BlockSpec((_BATCH, 1), lambda g: (0, 0)),
        out_shape=jax.ShapeDtypeStruct((_BATCH, 1), jnp.float32),
        scratch_shapes=[pltpu.VMEM((2, _BATCH), jnp.float32)],
    )(nodes, idx3, W, b2)
    return out


# speedup vs baseline: 14.8184x; 1.2396x over previous
import functools
import jax
import jax.numpy as jnp
from jax.experimental import pallas as pl
from jax.experimental.pallas import tpu as pltpu

_BATCH = 256


def _tc_body(nodes_ref, idx_ref, w_ref, b_ref, out_ref, acc_ref):
    g = pl.program_id(0)
    ng = pl.num_programs(0)

    @pl.when(g == 0)
    def _init():
        acc_ref[...] = jnp.zeros_like(acc_ref)

    blk = nodes_ref[...]
    y = jnp.dot(blk, w_ref[...], preferred_element_type=jnp.float32)
    idx = idx_ref[0, 0, :]
    onehot = (idx[:, None] == jax.lax.broadcasted_iota(
        jnp.int32, (1, _BATCH), 1)).astype(jnp.float32)
    y1 = jnp.concatenate([y, jnp.ones_like(y)], axis=1)
    acc_ref[...] += jax.lax.dot_general(
        y1, onehot, dimension_numbers=(((0,), (0,)), ((), ())),
        preferred_element_type=jnp.float32)

    @pl.when(g == ng - 1)
    def _final():
        ys = acc_ref[0, :]
        cnt = acc_ref[1, :]
        logits = ys / jnp.maximum(cnt, 1.0) + b_ref[0, 0]
        out_ref[...] = jax.nn.sigmoid(logits)[:, None]


def kernel(nodes, edges, receivers, senders, global_latent, node_graph_idx,
           edge_graph_idx, W, b):
    n, d = nodes.shape
    br = 10000
    grid = n // br
    idx3 = node_graph_idx.reshape(grid, 1, br)
    b2 = b.reshape(1, 1)

    out = pl.pallas_call(
        _tc_body,
        grid=(grid,),
        in_specs=[
            pl.BlockSpec((br, d), lambda g: (g, 0)),
            pl.BlockSpec((1, 1, br), lambda g: (g, 0, 0)),
            pl.BlockSpec((d, 1), lambda g: (0, 0)),
            pl.BlockSpec((1, 1), lambda g: (0, 0)),
        ],
        out_specs=pl.BlockSpec((_BATCH, 1), lambda g: (0, 0)),
        out_shape=jax.ShapeDtypeStruct((_BATCH, 1), jnp.float32),
        scratch_shapes=[pltpu.VMEM((2, _BATCH), jnp.float32)],
    )(nodes, idx3, W, b2)
    return out


# speedup vs baseline: 15.1476x; 1.0222x over previous
import functools
import jax
import jax.numpy as jnp
from jax.experimental import pallas as pl
from jax.experimental.pallas import tpu as pltpu

_BATCH = 256


def _tc_body(nodes_ref, idx_ref, w_ref, b_ref, out_ref, acc_ref):
    g = pl.program_id(0)
    ng = pl.num_programs(0)

    @pl.when(g == 0)
    def _init():
        acc_ref[...] = jnp.zeros_like(acc_ref)

    blk = nodes_ref[...]
    y = jnp.dot(blk, w_ref[...], preferred_element_type=jnp.float32)
    idx = idx_ref[0, 0, :]
    onehot = (idx[:, None] == jax.lax.broadcasted_iota(
        jnp.int32, (1, _BATCH), 1)).astype(jnp.float32)
    y1 = jnp.concatenate([y, jnp.ones_like(y)], axis=1)
    acc_ref[...] += jax.lax.dot_general(
        y1, onehot, dimension_numbers=(((0,), (0,)), ((), ())),
        preferred_element_type=jnp.float32)

    @pl.when(g == ng - 1)
    def _final():
        ys = acc_ref[0, :]
        cnt = acc_ref[1, :]
        logits = ys / jnp.maximum(cnt, 1.0) + b_ref[0, 0]
        out_ref[...] = jax.nn.sigmoid(logits)[:, None]


def kernel(nodes, edges, receivers, senders, global_latent, node_graph_idx,
           edge_graph_idx, W, b):
    n, d = nodes.shape
    br = 20000
    grid = n // br
    idx3 = node_graph_idx.reshape(grid, 1, br)
    b2 = b.reshape(1, 1)

    out = pl.pallas_call(
        _tc_body,
        grid=(grid,),
        in_specs=[
            pl.BlockSpec((br, d), lambda g: (g, 0)),
            pl.BlockSpec((1, 1, br), lambda g: (g, 0, 0)),
            pl.BlockSpec((d, 1), lambda g: (0, 0)),
            pl.BlockSpec((1, 1), lambda g: (0, 0)),
        ],
        out_specs=pl.BlockSpec((_BATCH, 1), lambda g: (0, 0)),
        out_shape=jax.ShapeDtypeStruct((_BATCH, 1), jnp.float32),
        scratch_shapes=[pltpu.VMEM((2, _BATCH), jnp.float32)],
    )(nodes, idx3, W, b2)
    return out
